# Initial kernel scaffold; baseline (speedup 1.0000x reference)
import jax
import jax.numpy as jnp
from jax.experimental import pallas as pl


def kernel(pos, rand_noise, flip_u):
    raise NotImplementedError("write your pallas kernel here")


# speedup vs baseline: 25.9920x; 25.9920x over previous
import functools
import jax
import jax.numpy as jnp
from jax.experimental import pallas as pl

_R2 = 0.1 * 0.1
_N = 10000
_NPAD = 10240
_TILE = 256
_GRID = _NPAD // _TILE
_BISECT_ITERS = 34
_SWEEPS = 12
_TOL2 = float(jnp.float32(1e-5) ** 2)


def _bf16(x):
    return x.astype(jnp.bfloat16).astype(jnp.float32)


def _dist_cov_kernel(pos_tile_ref, posT_ref, out_ref):
    pos_tile = pos_tile_ref[...]
    posT = posT_ref[...]
    p0 = posT[0:1, :]; p1 = posT[1:2, :]; p2 = posT[2:3, :]
    sq_j = (p0 * p0 + p2 * p2) + p1 * p1
    q0 = pos_tile[:, 0:1]; q1 = pos_tile[:, 1:2]; q2 = pos_tile[:, 2:3]
    sq_i = (q0 * q0 + q2 * q2) + q1 * q1
    g = jax.lax.dot_general(pos_tile, posT,
                            (((1,), (0,)), ((), ())))
    d2 = jnp.maximum(sq_i + sq_j - 2.0 * g, 0.0)
    within = d2 <= _R2
    cnt = jnp.sum(jnp.where(within, 1.0, 0.0), axis=1, keepdims=True)

    def bis_body(_, lohi):
        lo, hi = lohi
        mid = (lo + hi) * 0.5
        c = jnp.sum(jnp.where(within & (d2 <= mid), 1.0, 0.0), axis=1,
                    keepdims=True)
        ge = c >= 64.0
        hi = jnp.where(ge, mid, hi)
        lo = jnp.where(ge, lo, mid)
        return lo, hi
    lo0 = jnp.zeros_like(cnt)
    hi0 = jnp.full_like(cnt, _R2)
    _, hi = jax.lax.fori_loop(0, _BISECT_ITERS, bis_body, (lo0, hi0))
    thresh = jnp.where(cnt > 64.0, hi, jnp.full_like(hi, _R2))
    sel = within & (d2 <= thresh)
    self_mask = sel.astype(jnp.float32)

    vecs = []
    qvm = []
    qv = []
    for a in range(3):
        va = posT[a:a + 1, :] - pos_tile[:, a:a + 1]
        qv_a = _bf16(va)
        qvm_a = qv_a * self_mask
        qv.append(qv_a)
        qvm.append(qvm_a)
    out = []
    for a in range(3):
        for b in range(3):
            out.append(jnp.sum(qvm[a] * qv[b], axis=1, keepdims=True))
    out.append(cnt)
    for _ in range(6):
        out.append(jnp.zeros_like(cnt))
    out_ref[...] = jnp.concatenate(out, axis=1)


def _rot(A, V, p, q):
    app = A[(p, p)]; aqq = A[(q, q)]; apq = A[(p, q)]
    tau = (aqq - app) / (2.0 * apq)
    sq = jnp.sqrt(1.0 + tau * tau)
    t = jnp.where(tau >= 0, 1.0 / (tau + sq), 1.0 / (tau - sq))
    c = jax.lax.rsqrt(1.0 + t * t)
    s = t * c
    zero = apq == 0.0
    c = jnp.where(zero, 1.0, c)
    s = jnp.where(zero, 0.0, s)
    t = jnp.where(zero, 0.0, t)
    r = 3 - p - q
    a_rp = A[(min(r, p), max(r, p))]
    a_rq = A[(min(r, q), max(r, q))]
    nrp = c * a_rp - s * a_rq
    nrq = s * a_rp + c * a_rq
    A2 = dict(A)
    A2[(min(r, p), max(r, p))] = nrp
    A2[(min(r, q), max(r, q))] = nrq
    A2[(p, p)] = app - t * apq
    A2[(q, q)] = aqq + t * apq
    A2[(p, q)] = jnp.zeros_like(apq)
    V2 = dict(V)
    for i in range(3):
        v_p = V[(i, p)]; v_q = V[(i, q)]
        V2[(i, p)] = c * v_p - s * v_q
        V2[(i, q)] = s * v_p + c * v_q
    return A2, V2


def _eigh_tile(cov):
    A = {k: v for k, v in cov.items()}
    one = jnp.ones_like(A[(0, 0)])
    zero = jnp.zeros_like(one)
    V = {}
    for i in range(3):
        for j in range(3):
            V[(i, j)] = one if i == j else zero

    def off2(A):
        return (A[(0, 1)] * A[(0, 1)] + A[(0, 2)] * A[(0, 2)]
                + A[(1, 2)] * A[(1, 2)]) * 2.0

    def fro2(A):
        s = A[(0, 0)] * A[(0, 0)]
        s = s + A[(0, 1)] * A[(0, 1)] * 2.0
        s = s + A[(0, 2)] * A[(0, 2)] * 2.0
        s = s + A[(1, 1)] * A[(1, 1)]
        s = s + A[(1, 2)] * A[(1, 2)] * 2.0
        s = s + A[(2, 2)] * A[(2, 2)]
        return s

    def body(_, st):
        A, V = st
        active = off2(A) > _TOL2 * fro2(A)
        An, Vn = A, V
        for (p, q) in [(0, 2), (1, 2), (0, 1)]:
            An, Vn = _rot(An, Vn, p, q)
        A2 = {k: jnp.where(active, An[k], A[k]) for k in A}
        V2 = {k: jnp.where(active, Vn[k], V[k]) for k in V}
        return A2, V2

    keysA = sorted(A.keys())
    keysV = sorted(V.keys())

    def body_packed(i, packed):
        A = dict(zip(keysA, packed[:len(keysA)]))
        V = dict(zip(keysV, packed[len(keysA):]))
        A, V = body(i, (A, V))
        return tuple(A[k] for k in keysA) + tuple(V[k] for k in keysV)

    packed = tuple(A[k] for k in keysA) + tuple(V[k] for k in keysV)
    packed = jax.lax.fori_loop(0, _SWEEPS, body_packed, packed)
    A = dict(zip(keysA, packed[:len(keysA)]))
    V = dict(zip(keysV, packed[len(keysA):]))

    w = [A[(0, 0)], A[(1, 1)], A[(2, 2)]]
    cols = [[V[(i, 0)] for i in range(3)],
            [V[(i, 1)] for i in range(3)],
            [V[(i, 2)] for i in range(3)]]

    def cswap(w, cols, i, j):
        swp = w[j] < w[i]
        wi = jnp.where(swp, w[j], w[i]); wj = jnp.where(swp, w[i], w[j])
        w[i], w[j] = wi, wj
        ci = [jnp.where(swp, cols[j][k], cols[i][k]) for k in range(3)]
        cj = [jnp.where(swp, cols[i][k], cols[j][k]) for k in range(3)]
        cols[i], cols[j] = ci, cj
        return w, cols
    w, cols = cswap(w, cols, 0, 1)
    w, cols = cswap(w, cols, 1, 2)
    w, cols = cswap(w, cols, 0, 1)
    return cols


def _qr_tile(n):
    def house(x0, x1, x2):
        xn2 = x1 * x1 + x2 * x2
        mu = jnp.sqrt(x0 * x0 + xn2)
        beta = jnp.where(x0 <= 0, mu, -mu)
        tau = (beta - x0) / beta
        scale = 1.0 / (x0 - beta)
        v1 = x1 * scale; v2 = x2 * scale
        z = xn2 == 0.0
        tau = jnp.where(z, 0.0, tau)
        v1 = jnp.where(z, 0.0, v1)
        v2 = jnp.where(z, 0.0, v2)
        return v1, v2, tau
    a = {k: v for k, v in n.items()}
    v1, v2, tau0 = house(a[(0, 0)], a[(1, 0)], a[(2, 0)])

    def apply3(a, v1, v2, tau, cols):
        for j in cols:
            s = a[(0, j)] + v1 * a[(1, j)] + v2 * a[(2, j)]
            s = tau * s
            a[(0, j)] = a[(0, j)] - s
            a[(1, j)] = a[(1, j)] - v1 * s
            a[(2, j)] = a[(2, j)] - v2 * s
        return a

    def apply2(a, u1, tau, cols):
        for j in cols:
            s = a[(1, j)] + u1 * a[(2, j)]
            s = tau * s
            a[(1, j)] = a[(1, j)] - s
            a[(2, j)] = a[(2, j)] - u1 * s
        return a

    a = apply3(a, v1, v2, tau0, [0, 1, 2])
    zero = jnp.zeros_like(v1)
    u1, _, tau1 = house(a[(1, 1)], a[(2, 1)], zero)
    one = jnp.ones_like(v1)
    q = {}
    for i in range(3):
        for j in range(3):
            q[(i, j)] = one if i == j else zero
    q = apply2(q, u1, tau1, [0, 1, 2])
    q = apply3(q, v1, v2, tau0, [0, 1, 2])
    return q


def _frames_kernel(cov_ref, noise_ref, flip_ref, out_ref):
    cv = cov_ref[...]
    nz = noise_ref[...]
    fl = flip_ref[...]
    cov = {}
    for i in range(3):
        for j in range(i, 3):
            cov[(i, j)] = (cv[3 * i + j] + cv[3 * j + i]) * 0.5
    cnt = cv[9]
    cols = _eigh_tile(cov)
    n = {(i, j): nz[3 * i + j] for i in range(3) for j in range(3)}
    q = _qr_tile(n)
    flip = fl[0] < 0.5
    for j in range(3):
        q[(0, j)] = jnp.where(flip, -q[(0, j)], q[(0, j)])
    no_nb = cnt <= 1.0
    outs = []
    for i in range(3):
        for j in range(3):
            e = cols[i][j]
            o = jnp.where(no_nb, q[(j, i)], e)
            outs.append(o)
    out_ref[...] = jnp.stack(outs, axis=0)


@jax.jit
def kernel(pos, rand_noise, flip_u):
    pos_pad = jnp.concatenate(
        [pos, jnp.full((_NPAD - _N, 3), 100.0, jnp.float32)], axis=0)
    posT = pos_pad.T

    cov16 = pl.pallas_call(
        _dist_cov_kernel,
        grid=(_GRID,),
        in_specs=[
            pl.BlockSpec((_TILE, 3), lambda i: (i, 0)),
            pl.BlockSpec((3, _NPAD), lambda i: (0, 0)),
        ],
        out_specs=pl.BlockSpec((_TILE, 16), lambda i: (i, 0)),
        out_shape=jax.ShapeDtypeStruct((_NPAD, 16), jnp.float32),
    )(pos_pad, posT)

    H, W = _NPAD // 128, 128
    covT = cov16.T.reshape(16, H, W)
    noise_pad = jnp.concatenate(
        [rand_noise.reshape(_N, 9),
         jnp.zeros((_NPAD - _N, 9), jnp.float32)], axis=0)
    noiseT = noise_pad.T.reshape(9, H, W)
    flip_pad = jnp.concatenate(
        [flip_u, jnp.ones((_NPAD - _N,), jnp.float32)]).reshape(1, H, W)

    out9 = pl.pallas_call(
        _frames_kernel,
        out_shape=jax.ShapeDtypeStruct((9, H, W), jnp.float32),
    )(covT, noiseT, flip_pad)

    out = out9.reshape(9, _NPAD).T[:_N].reshape(_N, 3, 3)
    return out


# speedup vs baseline: 36.0401x; 1.3866x over previous
import functools
import numpy as np
import jax
import jax.numpy as jnp
from jax.experimental import pallas as pl

_R2 = 0.1 * 0.1
_N = 10000
_NPAD = 10240
_TILE = 256
_GRID = _NPAD // _TILE
_BISECT_ITERS = 34
_SWEEPS = 12
_TOL2 = float(np.float32(1e-5) * np.float32(1e-5))


def _bf16(x):
    return x.astype(jnp.bfloat16).astype(jnp.float32)


_CHUNK = 1024
_NCHUNK = _NPAD // _CHUNK


def _dist_cov_kernel(pos_tile_ref, posT_ref, out_ref, d2_ref):
    pos_tile = pos_tile_ref[...]
    q0 = pos_tile[:, 0:1]; q1 = pos_tile[:, 1:2]; q2 = pos_tile[:, 2:3]
    sq_i = (q0 * q0 + q2 * q2) + q1 * q1

    cnt = jnp.zeros((_TILE, 1), jnp.float32)
    for k in range(_NCHUNK):
        pT = posT_ref[:, k * _CHUNK:(k + 1) * _CHUNK]
        p0 = pT[0:1, :]; p1 = pT[1:2, :]; p2 = pT[2:3, :]
        sq_j = (p0 * p0 + p2 * p2) + p1 * p1
        g = jax.lax.dot_general(pos_tile, pT, (((1,), (0,)), ((), ())))
        d2c = jnp.maximum(sq_i + sq_j - 2.0 * g, 0.0)
        d2_ref[:, k * _CHUNK:(k + 1) * _CHUNK] = d2c
        cnt = cnt + jnp.sum(jnp.where(d2c <= _R2, 1.0, 0.0), axis=1,
                            keepdims=True)

    def bis_body(_, lohi):
        lo, hi = lohi
        mid = (lo + hi) * 0.5
        c = jnp.zeros((_TILE, 1), jnp.float32)
        for k in range(_NCHUNK):
            d2c = d2_ref[:, k * _CHUNK:(k + 1) * _CHUNK]
            c = c + jnp.sum(jnp.where(d2c <= mid, 1.0, 0.0), axis=1,
                            keepdims=True)
        ge = c >= 64.0
        hi = jnp.where(ge, mid, hi)
        lo = jnp.where(ge, lo, mid)
        return lo, hi
    lo0 = jnp.zeros_like(cnt)
    hi0 = jnp.full_like(cnt, _R2)
    _, hi = jax.lax.fori_loop(0, _BISECT_ITERS, bis_body, (lo0, hi0))
    thresh = jnp.where(cnt > 64.0, hi, jnp.full_like(hi, _R2))

    acc = [jnp.zeros((_TILE, 1), jnp.float32) for _ in range(6)]
    pairs = [(0, 0), (0, 1), (0, 2), (1, 1), (1, 2), (2, 2)]
    for k in range(_NCHUNK):
        pT = posT_ref[:, k * _CHUNK:(k + 1) * _CHUNK]
        d2c = d2_ref[:, k * _CHUNK:(k + 1) * _CHUNK]
        m = jnp.where(d2c <= thresh, 1.0, 0.0)
        qv = []
        for a in range(3):
            va = pT[a:a + 1, :] - pos_tile[:, a:a + 1]
            qv.append(_bf16(va))
        qm = [qv[a] * m for a in range(3)]
        for i, (a, b) in enumerate(pairs):
            acc[i] = acc[i] + jnp.sum(qm[a] * qv[b], axis=1, keepdims=True)
    covd = {(0, 0): acc[0], (0, 1): acc[1], (0, 2): acc[2],
            (1, 1): acc[3], (1, 2): acc[4], (2, 2): acc[5]}
    out = []
    for a in range(3):
        for b in range(3):
            out.append(covd[(min(a, b), max(a, b))])
    out.append(cnt)
    for _ in range(6):
        out.append(jnp.zeros_like(cnt))
    out_ref[...] = jnp.concatenate(out, axis=1)


def _rot(A, V, p, q):
    app = A[(p, p)]; aqq = A[(q, q)]; apq = A[(p, q)]
    tau = (aqq - app) / (2.0 * apq)
    sq = jnp.sqrt(1.0 + tau * tau)
    t = jnp.where(tau >= 0, 1.0 / (tau + sq), 1.0 / (tau - sq))
    c = jax.lax.rsqrt(1.0 + t * t)
    s = t * c
    zero = apq == 0.0
    c = jnp.where(zero, 1.0, c)
    s = jnp.where(zero, 0.0, s)
    t = jnp.where(zero, 0.0, t)
    r = 3 - p - q
    a_rp = A[(min(r, p), max(r, p))]
    a_rq = A[(min(r, q), max(r, q))]
    nrp = c * a_rp - s * a_rq
    nrq = s * a_rp + c * a_rq
    A2 = dict(A)
    A2[(min(r, p), max(r, p))] = nrp
    A2[(min(r, q), max(r, q))] = nrq
    A2[(p, p)] = app - t * apq
    A2[(q, q)] = aqq + t * apq
    A2[(p, q)] = jnp.zeros_like(apq)
    V2 = dict(V)
    for i in range(3):
        v_p = V[(i, p)]; v_q = V[(i, q)]
        V2[(i, p)] = c * v_p - s * v_q
        V2[(i, q)] = s * v_p + c * v_q
    return A2, V2


def _eigh_tile(cov):
    A = {k: v for k, v in cov.items()}
    one = jnp.ones_like(A[(0, 0)])
    zero = jnp.zeros_like(one)
    V = {}
    for i in range(3):
        for j in range(3):
            V[(i, j)] = one if i == j else zero

    def off2(A):
        return (A[(0, 1)] * A[(0, 1)] + A[(0, 2)] * A[(0, 2)]
                + A[(1, 2)] * A[(1, 2)]) * 2.0

    def fro2(A):
        s = A[(0, 0)] * A[(0, 0)]
        s = s + A[(0, 1)] * A[(0, 1)] * 2.0
        s = s + A[(0, 2)] * A[(0, 2)] * 2.0
        s = s + A[(1, 1)] * A[(1, 1)]
        s = s + A[(1, 2)] * A[(1, 2)] * 2.0
        s = s + A[(2, 2)] * A[(2, 2)]
        return s

    def body(_, st):
        A, V = st
        active = off2(A) > _TOL2 * fro2(A)
        An, Vn = A, V
        for (p, q) in [(0, 2), (1, 2), (0, 1)]:
            An, Vn = _rot(An, Vn, p, q)
        A2 = {k: jnp.where(active, An[k], A[k]) for k in A}
        V2 = {k: jnp.where(active, Vn[k], V[k]) for k in V}
        return A2, V2

    keysA = sorted(A.keys())
    keysV = sorted(V.keys())

    def body_packed(i, packed):
        A = dict(zip(keysA, packed[:len(keysA)]))
        V = dict(zip(keysV, packed[len(keysA):]))
        A, V = body(i, (A, V))
        return tuple(A[k] for k in keysA) + tuple(V[k] for k in keysV)

    packed = tuple(A[k] for k in keysA) + tuple(V[k] for k in keysV)
    packed = jax.lax.fori_loop(0, _SWEEPS, body_packed, packed)
    A = dict(zip(keysA, packed[:len(keysA)]))
    V = dict(zip(keysV, packed[len(keysA):]))

    w = [A[(0, 0)], A[(1, 1)], A[(2, 2)]]
    cols = [[V[(i, 0)] for i in range(3)],
            [V[(i, 1)] for i in range(3)],
            [V[(i, 2)] for i in range(3)]]

    def cswap(w, cols, i, j):
        swp = w[j] < w[i]
        wi = jnp.where(swp, w[j], w[i]); wj = jnp.where(swp, w[i], w[j])
        w[i], w[j] = wi, wj
        ci = [jnp.where(swp, cols[j][k], cols[i][k]) for k in range(3)]
        cj = [jnp.where(swp, cols[i][k], cols[j][k]) for k in range(3)]
        cols[i], cols[j] = ci, cj
        return w, cols
    w, cols = cswap(w, cols, 0, 1)
    w, cols = cswap(w, cols, 1, 2)
    w, cols = cswap(w, cols, 0, 1)
    return cols


def _qr_tile(n):
    def house(x0, x1, x2):
        xn2 = x1 * x1 + x2 * x2
        mu = jnp.sqrt(x0 * x0 + xn2)
        beta = jnp.where(x0 <= 0, mu, -mu)
        tau = (beta - x0) / beta
        scale = 1.0 / (x0 - beta)
        v1 = x1 * scale; v2 = x2 * scale
        z = xn2 == 0.0
        tau = jnp.where(z, 0.0, tau)
        v1 = jnp.where(z, 0.0, v1)
        v2 = jnp.where(z, 0.0, v2)
        return v1, v2, tau
    a = {k: v for k, v in n.items()}
    v1, v2, tau0 = house(a[(0, 0)], a[(1, 0)], a[(2, 0)])

    def apply3(a, v1, v2, tau, cols):
        for j in cols:
            s = a[(0, j)] + v1 * a[(1, j)] + v2 * a[(2, j)]
            s = tau * s
            a[(0, j)] = a[(0, j)] - s
            a[(1, j)] = a[(1, j)] - v1 * s
            a[(2, j)] = a[(2, j)] - v2 * s
        return a

    def apply2(a, u1, tau, cols):
        for j in cols:
            s = a[(1, j)] + u1 * a[(2, j)]
            s = tau * s
            a[(1, j)] = a[(1, j)] - s
            a[(2, j)] = a[(2, j)] - u1 * s
        return a

    a = apply3(a, v1, v2, tau0, [0, 1, 2])
    zero = jnp.zeros_like(v1)
    u1, _, tau1 = house(a[(1, 1)], a[(2, 1)], zero)
    one = jnp.ones_like(v1)
    q = {}
    for i in range(3):
        for j in range(3):
            q[(i, j)] = one if i == j else zero
    q = apply2(q, u1, tau1, [0, 1, 2])
    q = apply3(q, v1, v2, tau0, [0, 1, 2])
    return q


def _frames_kernel(cov_ref, noise_ref, flip_ref, out_ref):
    cv = cov_ref[...]
    nz = noise_ref[...]
    fl = flip_ref[...]
    cov = {}
    for i in range(3):
        for j in range(i, 3):
            cov[(i, j)] = (cv[3 * i + j] + cv[3 * j + i]) * 0.5
    cnt = cv[9]
    cols = _eigh_tile(cov)
    n = {(i, j): nz[3 * i + j] for i in range(3) for j in range(3)}
    q = _qr_tile(n)
    flip = fl[0] < 0.5
    for j in range(3):
        q[(0, j)] = jnp.where(flip, -q[(0, j)], q[(0, j)])
    no_nb = cnt <= 1.0
    outs = []
    for i in range(3):
        for j in range(3):
            e = cols[i][j]
            o = jnp.where(no_nb, q[(j, i)], e)
            outs.append(o)
    out_ref[...] = jnp.stack(outs, axis=0)


@jax.jit
def kernel(pos, rand_noise, flip_u):
    pos_pad = jnp.concatenate(
        [pos, jnp.full((_NPAD - _N, 3), 100.0, jnp.float32)], axis=0)
    posT = pos_pad.T

    from jax.experimental.pallas import tpu as pltpu
    cov16 = pl.pallas_call(
        _dist_cov_kernel,
        grid=(_GRID,),
        in_specs=[
            pl.BlockSpec((_TILE, 3), lambda i: (i, 0)),
            pl.BlockSpec((3, _NPAD), lambda i: (0, 0)),
        ],
        out_specs=pl.BlockSpec((_TILE, 16), lambda i: (i, 0)),
        out_shape=jax.ShapeDtypeStruct((_NPAD, 16), jnp.float32),
        scratch_shapes=[pltpu.VMEM((_TILE, _NPAD), jnp.float32)],
    )(pos_pad, posT)

    H, W = _NPAD // 128, 128
    covT = cov16.T.reshape(16, H, W)
    noise_pad = jnp.concatenate(
        [rand_noise.reshape(_N, 9),
         jnp.zeros((_NPAD - _N, 9), jnp.float32)], axis=0)
    noiseT = noise_pad.T.reshape(9, H, W)
    flip_pad = jnp.concatenate(
        [flip_u, jnp.ones((_NPAD - _N,), jnp.float32)]).reshape(1, H, W)

    out9 = pl.pallas_call(
        _frames_kernel,
        out_shape=jax.ShapeDtypeStruct((9, H, W), jnp.float32),
    )(covT, noiseT, flip_pad)

    out = out9.reshape(9, _NPAD).T[:_N].reshape(_N, 3, 3)
    return out
